# baseline (speedup 1.0000x reference)
import functools

import jax
import jax.numpy as jnp
from jax import lax
from jax.experimental import pallas as pl
from jax.experimental.pallas import tpu as pltpu
from jax.experimental.pallas import tpu_sc as plsc

EPS = 1e-5

NC = 2
NS = 16
NW = NC * NS
CH = 128


def _matmul_body(x_ref, w_ref, y_ref):
    y_ref[0] = jnp.dot(x_ref[...], w_ref[0], preferred_element_type=jnp.float32)


def _per_offset_matmul(x, w):
    K, Cin, Cout = w.shape
    N = x.shape[0]
    return pl.pallas_call(
        _matmul_body,
        grid=(K,),
        in_specs=[
            pl.BlockSpec((N, Cin), lambda k: (0, 0)),
            pl.BlockSpec((1, Cin, Cout), lambda k: (k, 0, 0)),
        ],
        out_specs=pl.BlockSpec((1, N, Cout), lambda k: (k, 0, 0)),
        out_shape=jax.ShapeDtypeStruct((K, N, Cout), jnp.float32),
    )(x, w)


def _bn_body(p_ref, g_ref, b_ref, o_ref, *, nrows):
    h = p_ref[0] + p_ref[1]
    mean = jnp.sum(h, axis=0, keepdims=True) / nrows
    var = jnp.sum(h * h, axis=0, keepdims=True) / nrows - mean * mean
    inv = lax.rsqrt(var + EPS)
    o_ref[...] = jnp.maximum((h - mean) * inv * g_ref[...] + b_ref[...], 0.0)


def _bn_res_body(p_ref, g_ref, b_ref, r_ref, o_ref, *, nrows):
    h = p_ref[0] + p_ref[1]
    mean = jnp.sum(h, axis=0, keepdims=True) / nrows
    var = jnp.sum(h * h, axis=0, keepdims=True) / nrows - mean * mean
    inv = lax.rsqrt(var + EPS)
    o_ref[...] = (
        jnp.maximum((h - mean) * inv * g_ref[...] + b_ref[...], 0.0) + r_ref[...]
    )


def _bn_relu(parts, gamma, beta, residual=None, *, n_nodes):
    N, C = n_nodes, parts.shape[2]
    gamma2 = gamma.reshape(1, C)
    beta2 = beta.reshape(1, C)
    p_spec = pl.BlockSpec((2, N, C), lambda i: (0, 0, 0))
    full = lambda s: pl.BlockSpec(s, lambda i: tuple(0 for _ in s))
    if residual is None:
        return pl.pallas_call(
            functools.partial(_bn_body, nrows=float(N)),
            grid=(1,),
            in_specs=[p_spec, full((1, C)), full((1, C))],
            out_specs=full((N, C)),
            out_shape=jax.ShapeDtypeStruct((N, C), jnp.float32),
        )(parts, gamma2, beta2)
    return pl.pallas_call(
        functools.partial(_bn_res_body, nrows=float(N)),
        grid=(1,),
        in_specs=[p_spec, full((1, C)), full((1, C)), full((N, C))],
        out_specs=full((N, C)),
        out_shape=jax.ShapeDtypeStruct((N, C), jnp.float32),
    )(parts, gamma2, beta2, residual)


def _sc_edge_accumulate(y_table, srcs, dsts, *, n_nodes, n_pad, cpt0, cpt1,
                        cols):
    cpt = max(cpt0, cpt1)
    rows_per_tile = n_pad // NS
    fl_rows = CH
    fl_chunks = rows_per_tile // fl_rows
    z_iters = n_pad // (NS * 16)

    mesh = plsc.VectorSubcoreMesh(core_axis_name="c", subcore_axis_name="s")

    @functools.partial(
        pl.kernel,
        out_type=jax.ShapeDtypeStruct((NC, n_pad, cols), jnp.float32),
        mesh=mesh,
        scratch_types=[
            pltpu.VMEM((cpt, CH), jnp.int32),
            pltpu.VMEM((cpt, CH), jnp.int32),
            pltpu.VMEM((CH, cols), jnp.float32),
            pltpu.VMEM((16, cols), jnp.float32),
            pltpu.VMEM_SHARED((n_pad, cols), jnp.float32),
            pltpu.SemaphoreType.DMA,
        ],
    )
    def run(y_hbm, src_hbm, dst_hbm, out_hbm, idx_s, idx_d, rows_v, zrow_v,
            acc_sh, sem):
        cid = lax.axis_index("c")
        sid = lax.axis_index("s")
        wid = sid * NC + cid

        zv = jnp.zeros((16,), jnp.float32)
        for i in range(16):
            for j in range(cols // 16):
                zrow_v[i, pl.ds(j * 16, 16)] = zv

        def zero_step(i, _):
            pltpu.sync_copy(
                zrow_v, acc_sh.at[pl.ds(sid * (n_pad // NS) + i * 16, 16)]
            )
            return 0
        lax.fori_loop(0, z_iters, zero_step, 0)

        pltpu.sync_copy(src_hbm.at[wid], idx_s)
        pltpu.sync_copy(dst_hbm.at[wid], idx_d)

        plsc.subcore_barrier()

        def edge_step(j, _):
            pltpu.async_copy(y_hbm.at[idx_s.at[j]], rows_v, sem).wait()
            pltpu.sync_copy(rows_v, acc_sh.at[idx_d.at[j]], add=True)
            return 0
        my_cpt = jnp.where(cid == 0, cpt0, cpt1)
        lax.fori_loop(0, my_cpt, edge_step, 0)

        plsc.subcore_barrier()

        def flush_step(c, _):
            base = sid * rows_per_tile + c * fl_rows
            pltpu.sync_copy(acc_sh.at[pl.ds(base, fl_rows)], rows_v)
            pltpu.sync_copy(rows_v, out_hbm.at[cid, pl.ds(base, fl_rows)])
            return 0
        lax.fori_loop(0, fl_chunks, flush_step, 0)

    return run(y_table, srcs, dsts)


def kernel(x_feat, w1, gamma1, beta1, w2, gamma2, beta2, edge_index):
    N, C = x_feat.shape
    K, E = edge_index.shape[1], edge_index.shape[2]
    e_tot = K * E
    cpt_tot = -(-e_tot // (NS * CH))
    cpt0 = (cpt_tot * 64) // 100
    cpt1 = cpt_tot - cpt0
    cpt_max = max(cpt0, cpt1)
    n_pad = ((N + 1 + NS * 16 - 1) // (NS * 16)) * (NS * 16)

    koff = (jnp.arange(K, dtype=jnp.int32) * N)[:, None]
    flat_src = (edge_index[0] + koff).reshape(-1)
    flat_dst = edge_index[1].reshape(-1)
    cap0 = NS * cpt0 * CH
    cap1 = NS * cpt1 * CH
    pad = cap0 + cap1 - e_tot
    flat_src = jnp.concatenate([flat_src, jnp.zeros((pad,), jnp.int32)])
    flat_dst = jnp.concatenate([flat_dst, jnp.full((pad,), N, jnp.int32)])

    def to_tiles(flat):
        p0 = flat[:cap0].reshape(NS, cpt0, CH)
        p1 = flat[cap0:].reshape(NS, cpt1, CH)
        p0 = jnp.pad(p0, ((0, 0), (0, cpt_max - cpt0), (0, 0)))
        p1 = jnp.pad(p1, ((0, 0), (0, cpt_max - cpt1), (0, 0)))
        return jnp.stack([p0, p1], axis=1).reshape(NW, cpt_max, CH)

    srcs = to_tiles(flat_src)
    dsts = to_tiles(flat_dst)

    def conv_block(feat, w, gamma, beta, residual):
        y = _per_offset_matmul(feat, w).reshape(K * N, C)
        parts = _sc_edge_accumulate(
            y, srcs, dsts, n_nodes=N, n_pad=n_pad, cpt0=cpt0, cpt1=cpt1,
            cols=C
        )
        return _bn_relu(parts, gamma, beta, residual, n_nodes=N)

    h1 = conv_block(x_feat, w1, gamma1, beta1, None)
    return conv_block(h1, w2, gamma2, beta2, x_feat)


# speedup vs baseline: 1.0332x; 1.0332x over previous
import functools

import jax
import jax.numpy as jnp
from jax import lax
from jax.experimental import pallas as pl
from jax.experimental.pallas import tpu as pltpu
from jax.experimental.pallas import tpu_sc as plsc

EPS = 1e-5

NC = 2
NS = 16
NW = NC * NS
CH = 128


def _matmul_body(x_ref, w_ref, y_ref):
    y_ref[0] = jnp.dot(x_ref[...], w_ref[0], preferred_element_type=jnp.float32)


def _per_offset_matmul(x, w):
    K, Cin, Cout = w.shape
    N = x.shape[0]
    return pl.pallas_call(
        _matmul_body,
        grid=(K,),
        in_specs=[
            pl.BlockSpec((N, Cin), lambda k: (0, 0)),
            pl.BlockSpec((1, Cin, Cout), lambda k: (k, 0, 0)),
        ],
        out_specs=pl.BlockSpec((1, N, Cout), lambda k: (k, 0, 0)),
        out_shape=jax.ShapeDtypeStruct((K, N, Cout), jnp.float32),
    )(x, w)


def _bn_body(p_ref, g_ref, b_ref, o_ref, *, nrows):
    h = p_ref[0] + p_ref[1]
    mean = jnp.sum(h, axis=0, keepdims=True) / nrows
    var = jnp.sum(h * h, axis=0, keepdims=True) / nrows - mean * mean
    inv = lax.rsqrt(var + EPS)
    o_ref[...] = jnp.maximum((h - mean) * inv * g_ref[...] + b_ref[...], 0.0)


def _bn_res_body(p_ref, g_ref, b_ref, r_ref, o_ref, *, nrows):
    h = p_ref[0] + p_ref[1]
    mean = jnp.sum(h, axis=0, keepdims=True) / nrows
    var = jnp.sum(h * h, axis=0, keepdims=True) / nrows - mean * mean
    inv = lax.rsqrt(var + EPS)
    o_ref[...] = (
        jnp.maximum((h - mean) * inv * g_ref[...] + b_ref[...], 0.0) + r_ref[...]
    )


def _bn_relu(parts, gamma, beta, residual=None, *, n_nodes):
    N, C = n_nodes, parts.shape[2]
    gamma2 = gamma.reshape(1, C)
    beta2 = beta.reshape(1, C)
    p_spec = pl.BlockSpec((2, N, C), lambda i: (0, 0, 0))
    full = lambda s: pl.BlockSpec(s, lambda i: tuple(0 for _ in s))
    if residual is None:
        return pl.pallas_call(
            functools.partial(_bn_body, nrows=float(N)),
            grid=(1,),
            in_specs=[p_spec, full((1, C)), full((1, C))],
            out_specs=full((N, C)),
            out_shape=jax.ShapeDtypeStruct((N, C), jnp.float32),
        )(parts, gamma2, beta2)
    return pl.pallas_call(
        functools.partial(_bn_res_body, nrows=float(N)),
        grid=(1,),
        in_specs=[p_spec, full((1, C)), full((1, C)), full((N, C))],
        out_specs=full((N, C)),
        out_shape=jax.ShapeDtypeStruct((N, C), jnp.float32),
    )(parts, gamma2, beta2, residual)


def _sc_edge_accumulate(y_table, srcs, dsts, *, n_nodes, n_pad, cpt0, cpt1,
                        cols):
    cpt = max(cpt0, cpt1)
    rows_per_tile = n_pad // NS
    fl_rows = CH
    fl_chunks = rows_per_tile // fl_rows
    z_iters = n_pad // (NS * 16)

    mesh = plsc.VectorSubcoreMesh(core_axis_name="c", subcore_axis_name="s")

    @functools.partial(
        pl.kernel,
        out_type=jax.ShapeDtypeStruct((NC, n_pad, cols), jnp.float32),
        mesh=mesh,
        scratch_types=[
            pltpu.VMEM((cpt, CH), jnp.int32),
            pltpu.VMEM((cpt, CH), jnp.int32),
            pltpu.VMEM((CH, cols), jnp.float32),
            pltpu.VMEM((16, cols), jnp.float32),
            pltpu.VMEM_SHARED((n_pad, cols), jnp.float32),
            pltpu.SemaphoreType.DMA,
        ],
    )
    def run(y_hbm, src_hbm, dst_hbm, out_hbm, idx_s, idx_d, rows_v, zrow_v,
            acc_sh, sem):
        cid = lax.axis_index("c")
        sid = lax.axis_index("s")
        wid = sid * NC + cid

        zv = jnp.zeros((16,), jnp.float32)
        for i in range(16):
            for j in range(cols // 16):
                zrow_v[i, pl.ds(j * 16, 16)] = zv

        def zero_step(i, _):
            pltpu.sync_copy(
                zrow_v, acc_sh.at[pl.ds(sid * (n_pad // NS) + i * 16, 16)]
            )
            return 0
        lax.fori_loop(0, z_iters, zero_step, 0)

        pltpu.sync_copy(src_hbm.at[wid], idx_s)
        pltpu.sync_copy(dst_hbm.at[wid], idx_d)

        plsc.subcore_barrier()

        def edge_step(j, _):
            pltpu.async_copy(y_hbm.at[idx_s.at[j]], rows_v, sem).wait()
            pltpu.sync_copy(rows_v, acc_sh.at[idx_d.at[j]], add=True)
            return 0
        my_cpt = jnp.where(cid == 0, cpt0, cpt1)
        lax.fori_loop(0, my_cpt, edge_step, 0)

        plsc.subcore_barrier()

        def flush_step(c, _):
            base = sid * rows_per_tile + c * fl_rows
            pltpu.sync_copy(acc_sh.at[pl.ds(base, fl_rows)], rows_v)
            pltpu.sync_copy(rows_v, out_hbm.at[cid, pl.ds(base, fl_rows)])
            return 0
        lax.fori_loop(0, fl_chunks, flush_step, 0)

    return run(y_table, srcs, dsts)


def kernel(x_feat, w1, gamma1, beta1, w2, gamma2, beta2, edge_index):
    N, C = x_feat.shape
    K, E = edge_index.shape[1], edge_index.shape[2]
    e_tot = K * E
    cpt_tot = -(-e_tot // (NS * CH))
    cpt0 = (cpt_tot * 61) // 100
    cpt1 = cpt_tot - cpt0
    cpt_max = max(cpt0, cpt1)
    n_pad = ((N + 1 + NS * 16 - 1) // (NS * 16)) * (NS * 16)

    koff = (jnp.arange(K, dtype=jnp.int32) * N)[:, None]
    flat_src = (edge_index[0] + koff).reshape(-1)
    flat_dst = edge_index[1].reshape(-1)
    cap0 = NS * cpt0 * CH
    cap1 = NS * cpt1 * CH
    pad = cap0 + cap1 - e_tot
    flat_src = jnp.concatenate([flat_src, jnp.zeros((pad,), jnp.int32)])
    flat_dst = jnp.concatenate([flat_dst, jnp.full((pad,), N, jnp.int32)])

    def to_tiles(flat):
        p0 = flat[:cap0].reshape(NS, cpt0, CH)
        p1 = flat[cap0:].reshape(NS, cpt1, CH)
        p0 = jnp.pad(p0, ((0, 0), (0, cpt_max - cpt0), (0, 0)))
        p1 = jnp.pad(p1, ((0, 0), (0, cpt_max - cpt1), (0, 0)))
        return jnp.stack([p0, p1], axis=1).reshape(NW, cpt_max, CH)

    srcs = to_tiles(flat_src)
    dsts = to_tiles(flat_dst)

    def conv_block(feat, w, gamma, beta, residual):
        y = _per_offset_matmul(feat, w).reshape(K * N, C)
        parts = _sc_edge_accumulate(
            y, srcs, dsts, n_nodes=N, n_pad=n_pad, cpt0=cpt0, cpt1=cpt1,
            cols=C
        )
        return _bn_relu(parts, gamma, beta, residual, n_nodes=N)

    h1 = conv_block(x_feat, w1, gamma1, beta1, None)
    return conv_block(h1, w2, gamma2, beta2, x_feat)


# speedup vs baseline: 1.0606x; 1.0266x over previous
import functools

import jax
import jax.numpy as jnp
from jax import lax
from jax.experimental import pallas as pl
from jax.experimental.pallas import tpu as pltpu
from jax.experimental.pallas import tpu_sc as plsc

EPS = 1e-5

NC = 2
NS = 16
NW = NC * NS
CH = 128


def _matmul_body(x_ref, w_ref, y_ref):
    y_ref[0] = jnp.dot(x_ref[...], w_ref[0], preferred_element_type=jnp.float32)


def _per_offset_matmul(x, w):
    K, Cin, Cout = w.shape
    N = x.shape[0]
    return pl.pallas_call(
        _matmul_body,
        grid=(K,),
        in_specs=[
            pl.BlockSpec((N, Cin), lambda k: (0, 0)),
            pl.BlockSpec((1, Cin, Cout), lambda k: (k, 0, 0)),
        ],
        out_specs=pl.BlockSpec((1, N, Cout), lambda k: (k, 0, 0)),
        out_shape=jax.ShapeDtypeStruct((K, N, Cout), jnp.float32),
    )(x, w)


def _bn_body(p_ref, g_ref, b_ref, o_ref, *, nrows):
    h = p_ref[0] + p_ref[1]
    mean = jnp.sum(h, axis=0, keepdims=True) / nrows
    var = jnp.sum(h * h, axis=0, keepdims=True) / nrows - mean * mean
    inv = lax.rsqrt(var + EPS)
    o_ref[...] = jnp.maximum((h - mean) * inv * g_ref[...] + b_ref[...], 0.0)


def _bn_res_body(p_ref, g_ref, b_ref, r_ref, o_ref, *, nrows):
    h = p_ref[0] + p_ref[1]
    mean = jnp.sum(h, axis=0, keepdims=True) / nrows
    var = jnp.sum(h * h, axis=0, keepdims=True) / nrows - mean * mean
    inv = lax.rsqrt(var + EPS)
    o_ref[...] = (
        jnp.maximum((h - mean) * inv * g_ref[...] + b_ref[...], 0.0) + r_ref[...]
    )


def _bn_relu(parts, gamma, beta, residual=None, *, n_nodes):
    N, C = n_nodes, parts.shape[2]
    gamma2 = gamma.reshape(1, C)
    beta2 = beta.reshape(1, C)
    p_spec = pl.BlockSpec((2, N, C), lambda i: (0, 0, 0))
    full = lambda s: pl.BlockSpec(s, lambda i: tuple(0 for _ in s))
    if residual is None:
        return pl.pallas_call(
            functools.partial(_bn_body, nrows=float(N)),
            grid=(1,),
            in_specs=[p_spec, full((1, C)), full((1, C))],
            out_specs=full((N, C)),
            out_shape=jax.ShapeDtypeStruct((N, C), jnp.float32),
        )(parts, gamma2, beta2)
    return pl.pallas_call(
        functools.partial(_bn_res_body, nrows=float(N)),
        grid=(1,),
        in_specs=[p_spec, full((1, C)), full((1, C)), full((N, C))],
        out_specs=full((N, C)),
        out_shape=jax.ShapeDtypeStruct((N, C), jnp.float32),
    )(parts, gamma2, beta2, residual)


def _sc_edge_accumulate(y_table, srcs, dsts, *, n_nodes, n_pad, cpt0, cpt1,
                        cols):
    cpt = max(cpt0, cpt1)
    rows_per_tile = n_pad // NS
    fl_rows = CH
    fl_chunks = rows_per_tile // fl_rows
    z_iters = n_pad // (NS * 16)

    mesh = plsc.VectorSubcoreMesh(core_axis_name="c", subcore_axis_name="s")

    @functools.partial(
        pl.kernel,
        out_type=jax.ShapeDtypeStruct((NC, n_pad, cols), jnp.float32),
        mesh=mesh,
        scratch_types=[
            pltpu.VMEM((cpt, CH), jnp.int32),
            pltpu.VMEM((cpt, CH), jnp.int32),
            pltpu.VMEM((CH, cols), jnp.float32),
            pltpu.VMEM((16, cols), jnp.float32),
            pltpu.VMEM_SHARED((n_pad, cols), jnp.float32),
            pltpu.SemaphoreType.DMA,
        ],
    )
    def run(y_hbm, src_hbm, dst_hbm, out_hbm, idx_s, idx_d, rows_v, zrow_v,
            acc_sh, sem):
        cid = lax.axis_index("c")
        sid = lax.axis_index("s")
        wid = sid * NC + cid

        zv = jnp.zeros((16,), jnp.float32)
        for i in range(16):
            for j in range(cols // 16):
                zrow_v[i, pl.ds(j * 16, 16)] = zv

        def zero_step(i, _):
            pltpu.sync_copy(
                zrow_v, acc_sh.at[pl.ds(sid * (n_pad // NS) + i * 16, 16)]
            )
            return 0
        lax.fori_loop(0, z_iters, zero_step, 0)

        pltpu.sync_copy(src_hbm.at[wid], idx_s)
        pltpu.sync_copy(dst_hbm.at[wid], idx_d)

        plsc.subcore_barrier()

        def edge_step(j, _):
            pltpu.async_copy(y_hbm.at[idx_s.at[j]], rows_v, sem).wait()
            pltpu.sync_copy(rows_v, acc_sh.at[idx_d.at[j]], add=True)
            return 0
        my_cpt = jnp.where(cid == 0, cpt0, cpt1)
        lax.fori_loop(0, my_cpt, edge_step, 0)

        plsc.subcore_barrier()

        def flush_step(c, _):
            base = sid * rows_per_tile + c * fl_rows
            pltpu.sync_copy(acc_sh.at[pl.ds(base, fl_rows)], rows_v)
            pltpu.sync_copy(rows_v, out_hbm.at[cid, pl.ds(base, fl_rows)])
            return 0
        lax.fori_loop(0, fl_chunks, flush_step, 0)

    return run(y_table, srcs, dsts)


def kernel(x_feat, w1, gamma1, beta1, w2, gamma2, beta2, edge_index):
    N, C = x_feat.shape
    K, E = edge_index.shape[1], edge_index.shape[2]
    e_tot = K * E
    cpt_tot = -(-e_tot // (NS * CH))
    cpt0 = (cpt_tot * 58) // 100
    cpt1 = cpt_tot - cpt0
    cpt_max = max(cpt0, cpt1)
    n_pad = ((N + 1 + NS * 16 - 1) // (NS * 16)) * (NS * 16)

    koff = (jnp.arange(K, dtype=jnp.int32) * N)[:, None]
    flat_src = (edge_index[0] + koff).reshape(-1)
    flat_dst = edge_index[1].reshape(-1)
    cap0 = NS * cpt0 * CH
    cap1 = NS * cpt1 * CH
    pad = cap0 + cap1 - e_tot
    flat_src = jnp.concatenate([flat_src, jnp.zeros((pad,), jnp.int32)])
    flat_dst = jnp.concatenate([flat_dst, jnp.full((pad,), N, jnp.int32)])

    def to_tiles(flat):
        p0 = flat[:cap0].reshape(NS, cpt0, CH)
        p1 = flat[cap0:].reshape(NS, cpt1, CH)
        p0 = jnp.pad(p0, ((0, 0), (0, cpt_max - cpt0), (0, 0)))
        p1 = jnp.pad(p1, ((0, 0), (0, cpt_max - cpt1), (0, 0)))
        return jnp.stack([p0, p1], axis=1).reshape(NW, cpt_max, CH)

    srcs = to_tiles(flat_src)
    dsts = to_tiles(flat_dst)

    def conv_block(feat, w, gamma, beta, residual):
        y = _per_offset_matmul(feat, w).reshape(K * N, C)
        parts = _sc_edge_accumulate(
            y, srcs, dsts, n_nodes=N, n_pad=n_pad, cpt0=cpt0, cpt1=cpt1,
            cols=C
        )
        return _bn_relu(parts, gamma, beta, residual, n_nodes=N)

    h1 = conv_block(x_feat, w1, gamma1, beta1, None)
    return conv_block(h1, w2, gamma2, beta2, x_feat)
